# baseline (speedup 1.0000x reference)
import jax
import jax.numpy as jnp
from jax.experimental import pallas as pl

QB = 256
KB = 256


def _flash_kernel(docs_col_ref, docs_row_ref, q_ref, k_ref, v_ref, o_ref):
    n = docs_row_ref.shape[1]
    d = q_ref.shape[-1]
    q = q_ref[0]
    docs_q = docs_col_ref[:, 0:1]
    d_row = docs_row_ref[0:1, :]

    qd_min = docs_q[0, 0]
    qd_max = docs_q[QB - 1, 0]
    lo = jnp.sum((d_row < qd_min).astype(jnp.int32))
    hi = jnp.sum((d_row <= qd_max).astype(jnp.int32))
    kb_lo = lo // KB
    kb_hi = (hi + KB - 1) // KB

    scale = 1.0 / (d ** 0.5)

    def body(kb, carry):
        m, l, acc = carry
        off = kb * KB
        k = k_ref[0, pl.ds(off, KB), :]
        v = v_ref[0, pl.ds(off, KB), :]
        docs_k = docs_row_ref[0:1, pl.ds(off, KB)]
        s = jax.lax.dot_general(q, k, (((1,), (1,)), ((), ())),
                                preferred_element_type=jnp.float32) * scale
        mask = docs_q == docs_k
        s = jnp.where(mask, s, -1e30)
        m_new = jnp.maximum(m, jnp.max(s, axis=1, keepdims=True))
        p = jnp.where(mask, jnp.exp(s - m_new), 0.0)
        alpha = jnp.exp(m - m_new)
        l_new = l * alpha + jnp.sum(p, axis=1, keepdims=True)
        acc_new = acc * alpha + jax.lax.dot_general(
            p, v, (((1,), (0,)), ((), ())), preferred_element_type=jnp.float32)
        return m_new, l_new, acc_new

    m0 = jnp.full((QB, 1), -1e30, jnp.float32)
    l0 = jnp.zeros((QB, 1), jnp.float32)
    acc0 = jnp.zeros((QB, d), jnp.float32)
    m, l, acc = jax.lax.fori_loop(kb_lo, kb_hi, body, (m0, l0, acc0))
    o_ref[0] = acc / l


def kernel(Q, K, V, document_id):
    b, h, n, d = Q.shape
    docs = document_id.astype(jnp.int32)
    Qr = Q.reshape(b * h, n, d)
    Kr = K.reshape(b * h, n, d)
    Vr = V.reshape(b * h, n, d)
    docs_col = jnp.broadcast_to(docs[:, None], (n, 128))
    docs_row = jnp.broadcast_to(docs[None, :], (8, n))

    out = pl.pallas_call(
        _flash_kernel,
        grid=(b * h, n // QB),
        in_specs=[
            pl.BlockSpec((QB, 128), lambda hh, i: (i, 0)),
            pl.BlockSpec((8, n), lambda hh, i: (0, 0)),
            pl.BlockSpec((1, QB, d), lambda hh, i: (hh, i, 0)),
            pl.BlockSpec((1, n, d), lambda hh, i: (hh, 0, 0)),
            pl.BlockSpec((1, n, d), lambda hh, i: (hh, 0, 0)),
        ],
        out_specs=pl.BlockSpec((1, QB, d), lambda hh, i: (hh, i, 0)),
        out_shape=jax.ShapeDtypeStruct((b * h, n, d), jnp.float32),
    )(docs_col, docs_row, Qr, Kr, Vr)
    return out.reshape(b, h, n, d)
